# baseline (speedup 1.0000x reference)
import jax
import jax.numpy as jnp
from jax.experimental import pallas as pl

INPUT = 784
HIDDEN = 8192
OUT = 10
BLK = 1024


def _fused_kernel(x_ref, w_ref, wsup_ref, b_ref, out_ref):
    i = pl.program_id(0)
    h = jax.lax.dot_general(
        w_ref[...], x_ref[...],
        (((1,), (1,)), ((), ())),
        preferred_element_type=jnp.float32,
    )
    h = jnp.maximum(h, 0.0)
    part = jax.lax.dot_general(
        wsup_ref[...], h,
        (((1,), (0,)), ((), ())),
        preferred_element_type=jnp.float32,
    )

    @pl.when(i == 0)
    def _():
        out_ref[...] = b_ref[...] + part

    @pl.when(i != 0)
    def _():
        out_ref[...] = out_ref[...] + part


def kernel(x, W_uns, W_sup, b_sup):
    x2 = x.reshape(1, INPUT)
    b2 = b_sup.reshape(OUT, 1)
    out = pl.pallas_call(
        _fused_kernel,
        grid=(HIDDEN // BLK,),
        in_specs=[
            pl.BlockSpec((1, INPUT), lambda i: (0, 0)),
            pl.BlockSpec((BLK, INPUT), lambda i: (i, 0)),
            pl.BlockSpec((OUT, BLK), lambda i: (0, i)),
            pl.BlockSpec((OUT, 1), lambda i: (0, 0)),
        ],
        out_specs=pl.BlockSpec((OUT, 1), lambda i: (0, 0)),
        out_shape=jax.ShapeDtypeStruct((OUT, 1), jnp.float32),
    )(x2, W_uns, W_sup, b2)
    return out.reshape(OUT)


# speedup vs baseline: 1.0181x; 1.0181x over previous
import jax
import jax.numpy as jnp
from jax.experimental import pallas as pl

INPUT = 784
HIDDEN = 8192
OUT = 10
BLK = 2048


def _fused_kernel(x_ref, w_ref, wsup_ref, b_ref, out_ref):
    i = pl.program_id(0)
    h = jax.lax.dot_general(
        w_ref[...], x_ref[...],
        (((1,), (1,)), ((), ())),
        preferred_element_type=jnp.float32,
    )
    h = jnp.maximum(h, 0.0)
    part = jax.lax.dot_general(
        wsup_ref[...], h,
        (((1,), (0,)), ((), ())),
        preferred_element_type=jnp.float32,
    )

    @pl.when(i == 0)
    def _():
        out_ref[...] = b_ref[...] + part

    @pl.when(i != 0)
    def _():
        out_ref[...] = out_ref[...] + part


def kernel(x, W_uns, W_sup, b_sup):
    x2 = x.reshape(1, INPUT)
    b2 = b_sup.reshape(OUT, 1)
    out = pl.pallas_call(
        _fused_kernel,
        grid=(HIDDEN // BLK,),
        in_specs=[
            pl.BlockSpec((1, INPUT), lambda i: (0, 0)),
            pl.BlockSpec((BLK, INPUT), lambda i: (i, 0)),
            pl.BlockSpec((OUT, BLK), lambda i: (0, i)),
            pl.BlockSpec((OUT, 1), lambda i: (0, 0)),
        ],
        out_specs=pl.BlockSpec((OUT, 1), lambda i: (0, 0)),
        out_shape=jax.ShapeDtypeStruct((OUT, 1), jnp.float32),
    )(x2, W_uns, W_sup, b2)
    return out.reshape(OUT)


# speedup vs baseline: 1.0300x; 1.0117x over previous
import jax
import jax.numpy as jnp
from jax.experimental import pallas as pl
from jax.experimental.pallas import tpu as pltpu

INPUT = 784
HIDDEN = 8192
OUT = 10
NCHUNK = 16
CHUNK = HIDDEN // NCHUNK
NBUF = 8


def _stream_kernel(x_ref, wsup_ref, b_ref, w_hbm, out_ref, wbuf, sems):
    def copy(i, slot):
        return pltpu.make_async_copy(
            w_hbm.at[pl.ds(i * CHUNK, CHUNK), :],
            wbuf.at[slot],
            sems.at[slot],
        )

    for s in range(NBUF):
        copy(s, s).start()

    acc = jnp.zeros((OUT, 1), jnp.float32)
    for i in range(NCHUNK):
        slot = i % NBUF
        copy(i, slot).wait()
        h = jax.lax.dot_general(
            wbuf[slot], x_ref[...],
            (((1,), (1,)), ((), ())),
            preferred_element_type=jnp.float32,
        )
        h = jnp.maximum(h, 0.0)
        acc = acc + jax.lax.dot_general(
            wsup_ref[:, i * CHUNK:(i + 1) * CHUNK], h,
            (((1,), (0,)), ((), ())),
            preferred_element_type=jnp.float32,
        )
        if i + NBUF < NCHUNK:
            copy(i + NBUF, slot).start()
    out_ref[...] = b_ref[...] + acc


def kernel(x, W_uns, W_sup, b_sup):
    x2 = x.reshape(1, INPUT)
    b2 = b_sup.reshape(OUT, 1)
    out = pl.pallas_call(
        _stream_kernel,
        in_specs=[
            pl.BlockSpec((1, INPUT), lambda: (0, 0)),
            pl.BlockSpec((OUT, HIDDEN), lambda: (0, 0)),
            pl.BlockSpec((OUT, 1), lambda: (0, 0)),
            pl.BlockSpec(memory_space=pltpu.MemorySpace.HBM),
        ],
        out_specs=pl.BlockSpec((OUT, 1), lambda: (0, 0)),
        out_shape=jax.ShapeDtypeStruct((OUT, 1), jnp.float32),
        scratch_shapes=[
            pltpu.VMEM((NBUF, CHUNK, INPUT), jnp.float32),
            pltpu.SemaphoreType.DMA((NBUF,)),
        ],
    )(x2, W_sup, b2, W_uns)
    return out.reshape(OUT)


# speedup vs baseline: 2.7975x; 2.7160x over previous
import jax
import jax.numpy as jnp
from jax.experimental import pallas as pl

INPUT = 784
HIDDEN = 8192
OUT = 10
BLK = 1024


def _fused_kernel(x_ref, wt_ref, wsup_ref, b_ref, out_ref):
    i = pl.program_id(0)
    h = jax.lax.dot_general(
        x_ref[...], wt_ref[...],
        (((1,), (0,)), ((), ())),
        preferred_element_type=jnp.float32,
    )
    h = jnp.maximum(h, 0.0)
    part = jax.lax.dot_general(
        wsup_ref[...], h,
        (((1,), (1,)), ((), ())),
        preferred_element_type=jnp.float32,
    )

    @pl.when(i == 0)
    def _():
        out_ref[...] = b_ref[...] + part

    @pl.when(i != 0)
    def _():
        out_ref[...] = out_ref[...] + part


def kernel(x, W_uns, W_sup, b_sup):
    x2 = x.reshape(1, INPUT)
    b2 = b_sup.reshape(OUT, 1)
    wt = W_uns.T
    out = pl.pallas_call(
        _fused_kernel,
        grid=(HIDDEN // BLK,),
        in_specs=[
            pl.BlockSpec((1, INPUT), lambda i: (0, 0)),
            pl.BlockSpec((INPUT, BLK), lambda i: (0, i)),
            pl.BlockSpec((OUT, BLK), lambda i: (0, i)),
            pl.BlockSpec((OUT, 1), lambda i: (0, 0)),
        ],
        out_specs=pl.BlockSpec((OUT, 1), lambda i: (0, 0)),
        out_shape=jax.ShapeDtypeStruct((OUT, 1), jnp.float32),
    )(x2, wt, W_sup, b2)
    return out.reshape(OUT)


# speedup vs baseline: 2.9464x; 1.0532x over previous
import jax
import jax.numpy as jnp
from jax.experimental import pallas as pl

INPUT = 784
HIDDEN = 8192
OUT = 10
BLK = 2048


def _fused_kernel(x_ref, wt_ref, wsup_ref, b_ref, out_ref):
    i = pl.program_id(0)
    h = jax.lax.dot_general(
        x_ref[...], wt_ref[...],
        (((1,), (0,)), ((), ())),
        preferred_element_type=jnp.float32,
    )
    h = jnp.maximum(h, 0.0)
    part = jax.lax.dot_general(
        wsup_ref[...], h,
        (((1,), (1,)), ((), ())),
        preferred_element_type=jnp.float32,
    )

    @pl.when(i == 0)
    def _():
        out_ref[...] = b_ref[...] + part

    @pl.when(i != 0)
    def _():
        out_ref[...] = out_ref[...] + part


def kernel(x, W_uns, W_sup, b_sup):
    x2 = x.reshape(1, INPUT)
    b2 = b_sup.reshape(OUT, 1)
    wt = W_uns.T
    out = pl.pallas_call(
        _fused_kernel,
        grid=(HIDDEN // BLK,),
        in_specs=[
            pl.BlockSpec((1, INPUT), lambda i: (0, 0)),
            pl.BlockSpec((INPUT, BLK), lambda i: (0, i)),
            pl.BlockSpec((OUT, BLK), lambda i: (0, i)),
            pl.BlockSpec((OUT, 1), lambda i: (0, 0)),
        ],
        out_specs=pl.BlockSpec((OUT, 1), lambda i: (0, 0)),
        out_shape=jax.ShapeDtypeStruct((OUT, 1), jnp.float32),
    )(x2, wt, W_sup, b2)
    return out.reshape(OUT)


# speedup vs baseline: 2.9872x; 1.0139x over previous
import jax
import jax.numpy as jnp
from jax.experimental import pallas as pl

INPUT = 784
HIDDEN = 8192
OUT = 10
BLK = 4096


def _fused_kernel(x_ref, wt_ref, wsup_ref, b_ref, out_ref):
    i = pl.program_id(0)
    h = jax.lax.dot_general(
        x_ref[...], wt_ref[...],
        (((1,), (0,)), ((), ())),
        preferred_element_type=jnp.float32,
    )
    h = jnp.maximum(h, 0.0)
    part = jax.lax.dot_general(
        wsup_ref[...], h,
        (((1,), (1,)), ((), ())),
        preferred_element_type=jnp.float32,
    )

    @pl.when(i == 0)
    def _():
        out_ref[...] = b_ref[...] + part

    @pl.when(i != 0)
    def _():
        out_ref[...] = out_ref[...] + part


def kernel(x, W_uns, W_sup, b_sup):
    x2 = x.reshape(1, INPUT)
    b2 = b_sup.reshape(OUT, 1)
    wt = W_uns.T
    out = pl.pallas_call(
        _fused_kernel,
        grid=(HIDDEN // BLK,),
        in_specs=[
            pl.BlockSpec((1, INPUT), lambda i: (0, 0)),
            pl.BlockSpec((INPUT, BLK), lambda i: (0, i)),
            pl.BlockSpec((OUT, BLK), lambda i: (0, i)),
            pl.BlockSpec((OUT, 1), lambda i: (0, 0)),
        ],
        out_specs=pl.BlockSpec((OUT, 1), lambda i: (0, 0)),
        out_shape=jax.ShapeDtypeStruct((OUT, 1), jnp.float32),
    )(x2, wt, W_sup, b2)
    return out.reshape(OUT)
